# Initial kernel scaffold; baseline (speedup 1.0000x reference)
import jax
import jax.numpy as jnp
from jax.experimental import pallas as pl


def kernel(x, edge_index, batch, wavelet_constructor, W_in, mlp_W1, mlp_b1, ln_g, ln_b, mlp_W2, mlp_b2):
    raise NotImplementedError("write your pallas kernel here")


# speedup vs baseline: 10.7549x; 10.7549x over previous
import jax
import jax.numpy as jnp
from jax import lax
from jax.experimental import pallas as pl
from jax.experimental.pallas import tpu as pltpu
from jax.experimental.pallas import tpu_sc as plsc

N = 10000
E = 320000
C = 128
L = 4
NP = 10240
NB = 160
W = 128
BR = 256
GRID = NP // BR

_mesh = plsc.VectorSubcoreMesh(core_axis_name="c", subcore_axis_name="s")

_f32 = jnp.float32
_CHECKPOINTS = ((1, 2), (1, 2, 4), (2, 4, 8), (4, 8, 16))


def _deg_body(col_hbm, out_hbm, cbuf, obuf, zbuf, degs):
    c = lax.axis_index("c")
    s = lax.axis_index("s")
    wid = c * 16 + s
    for j in range(8):
        obuf[pl.ds(j * 16, 16)] = jnp.full((16,), 1.0, _f32)

    def zfill(i, _):
        zbuf[pl.ds(i * 16, 16)] = jnp.full((16,), 0.0, _f32)
        return 0
    lax.fori_loop(0, (NP // 16) // 16, zfill, 0)
    pltpu.sync_copy(zbuf, degs.at[pl.ds(s * (NP // 16), NP // 16)])
    plsc.subcore_barrier()

    nwin = (E // W) // 32
    rem = (E // W) - nwin * 32
    w0 = wid * nwin + jnp.minimum(wid, rem)
    cnt = nwin + jnp.where(wid < rem, 1, 0)

    def body(j, _):
        base = (w0 + j) * W
        pltpu.sync_copy(col_hbm.at[pl.ds(base, W)], cbuf)
        pltpu.sync_copy(obuf, degs.at[cbuf], add=True)
        return 0
    lax.fori_loop(0, cnt, body, 0)
    plsc.subcore_barrier()
    pltpu.sync_copy(degs.at[pl.ds(s * (NP // 16), NP // 16)],
                    out_hbm.at[c, pl.ds(s * (NP // 16), NP // 16)])


_deg_call = pl.kernel(
    _deg_body,
    out_type=jax.ShapeDtypeStruct((2, NP), _f32),
    mesh=_mesh,
    scratch_types=[
        pltpu.VMEM((W,), jnp.int32),
        pltpu.VMEM((W,), _f32),
        pltpu.VMEM((NP // 16,), _f32),
        pltpu.VMEM_SHARED((NP,), _f32),
    ],
)


def _round_body(rs_hbm, cs_hbm, ts_hbm, dinvb_hbm, g_hbm, h_hbm,
                hout_hbm, gout_hbm,
                sbuf, ridx, cidx, vbuf, hbuf, abuf, dbuf, agg):
    c = lax.axis_index("c")
    s = lax.axis_index("s")

    pltpu.sync_copy(ts_hbm, sbuf)

    for p in range(2):
        blk = 32 * c + 16 * p + s
        n0 = blk * NB
        n0l = s * NB
        cb = c * (NP // 2) + p * (NP // 4)

        pltpu.sync_copy(dinvb_hbm.at[pl.ds(n0, NB)], dbuf)
        pltpu.sync_copy(h_hbm.at[pl.ds(n0, NB)], hbuf)

        def zfill(i, _):
            for k in range(8):
                abuf[i, pl.ds(k * 16, 16)] = jnp.full((16,), 0.0, _f32)
            return 0
        lax.fori_loop(0, NB, zfill, 0)
        pltpu.sync_copy(abuf, agg.at[pl.ds(n0l, NB)])

        p0 = sbuf[pl.ds(blk, 16)][0]
        p1 = sbuf[pl.ds(blk + 1, 16)][0]
        a0 = (p0 // 8) * 8
        nw = (p1 - a0 + (W - 1)) // W

        def win(j, _):
            base = a0 + j * W
            pltpu.sync_copy(rs_hbm.at[pl.ds(base, W)], ridx)
            pltpu.sync_copy(cs_hbm.at[pl.ds(base, W)], cidx)
            for t in range(8):
                ii = base + t * 16 + lax.iota(jnp.int32, 16)
                valid = (ii >= p0) & (ii < p1)
                dummy = jnp.full((16,), (NP // 4) + ((s * 8 + t) % 64),
                                 jnp.int32)
                cidx[pl.ds(t * 16, 16)] = jnp.where(
                    valid, cidx[pl.ds(t * 16, 16)] - cb, dummy)
            pltpu.sync_copy(g_hbm.at[ridx], vbuf)
            pltpu.sync_copy(vbuf, agg.at[cidx], add=True)
            return 0
        lax.fori_loop(0, nw, win, 0)

        pltpu.sync_copy(agg.at[pl.ds(n0l, NB)], abuf)

        def fin(i, _):
            for k in range(8):
                hv = hbuf[i, pl.ds(k * 16, 16)]
                av = abuf[i, pl.ds(k * 16, 16)]
                hbuf[i, pl.ds(k * 16, 16)] = 0.5 * (hv + av)
            return 0
        lax.fori_loop(0, NB, fin, 0)
        pltpu.sync_copy(hbuf, hout_hbm.at[pl.ds(n0, NB)])

        def gmul(i, _):
            for k in range(8):
                abuf[i, pl.ds(k * 16, 16)] = (
                    hbuf[i, pl.ds(k * 16, 16)] * dbuf[i, pl.ds(k * 16, 16)])
            return 0
        lax.fori_loop(0, NB, gmul, 0)
        pltpu.sync_copy(abuf, gout_hbm.at[pl.ds(n0, NB)])


_round_call = pl.kernel(
    _round_body,
    out_type=[jax.ShapeDtypeStruct((NP, C), _f32),
              jax.ShapeDtypeStruct((NP, C), _f32)],
    mesh=_mesh,
    scratch_types=[
        pltpu.VMEM((80,), jnp.int32),
        pltpu.VMEM((W,), jnp.int32),
        pltpu.VMEM((W,), jnp.int32),
        pltpu.VMEM((W, C), _f32),
        pltpu.VMEM((NB, C), _f32),
        pltpu.VMEM((NB, C), _f32),
        pltpu.VMEM((NB, C), _f32),
        pltpu.VMEM_SHARED((NP // 4 + 64, C), _f32),
    ],
)


def _input_body(x_ref, w_ref, dp_ref, di_ref, db_ref, h_ref, g_ref):
    d = dp_ref[0] + dp_ref[1]
    di = jnp.where(d > 0, 1.0 / d, 0.0)
    h0 = lax.dot_general(x_ref[...], w_ref[...],
                         (((1,), (1,)), ((), ())),
                         preferred_element_type=_f32)
    di_ref[...] = di
    db_ref[...] = jnp.broadcast_to(di, (BR, C))
    h_ref[...] = h0
    g_ref[...] = h0 * di


_input_call = pl.pallas_call(
    _input_body,
    grid=(GRID,),
    in_specs=[
        pl.BlockSpec((BR, C), lambda i: (i, 0)),
        pl.BlockSpec((C, C), lambda i: (0, 0)),
        pl.BlockSpec((2, BR, 1), lambda i: (0, i, 0)),
    ],
    out_specs=[
        pl.BlockSpec((BR, 1), lambda i: (i, 0)),
        pl.BlockSpec((BR, C), lambda i: (i, 0)),
        pl.BlockSpec((BR, C), lambda i: (i, 0)),
        pl.BlockSpec((BR, C), lambda i: (i, 0)),
    ],
    out_shape=[
        jax.ShapeDtypeStruct((NP, 1), _f32),
        jax.ShapeDtypeStruct((NP, C), _f32),
        jax.ShapeDtypeStruct((NP, C), _f32),
        jax.ShapeDtypeStruct((NP, C), _f32),
    ],
)


def _make_stage1(nterm):
    def body(*refs):
        d_refs = refs[:nterm]
        ce_ref, w1_ref, b1_ref = refs[nterm:nterm + 3]
        z_ref = refs[nterm + 3]
        def _b(v):
            return lax.convert_element_type(
                lax.convert_element_type(v, jnp.bfloat16), _f32)
        psi = _b(ce_ref[0]) * _b(d_refs[0][...])
        for t in range(1, nterm):
            psi = psi + _b(ce_ref[t]) * _b(d_refs[t][...])
        z_ref[...] = lax.dot_general(psi, w1_ref[...], (((1,), (1,)), ((), ())),
                                     preferred_element_type=_f32) + b1_ref[...]

    in_specs = [pl.BlockSpec((BR, C), lambda i: (i, 0))] * nterm
    in_specs.append(pl.BlockSpec(memory_space=pltpu.SMEM))
    in_specs += [
        pl.BlockSpec((C, C), lambda i: (0, 0)),
        pl.BlockSpec((1, C), lambda i: (0, 0)),
    ]
    return pl.pallas_call(
        body,
        grid=(GRID,),
        in_specs=in_specs,
        out_specs=pl.BlockSpec((BR, C), lambda i: (i, 0)),
        out_shape=jax.ShapeDtypeStruct((NP, C), _f32),
    )


_stage1_calls = {2: _make_stage1(2), 3: _make_stage1(3)}


def _make_stage2(want_g):
    def body(*refs):
        zg_ref, w2_ref, b2_ref = refs[:3]
        i = 3
        if want_g:
            di_ref = refs[i]; i += 1
        outs = refs[i:]
        hn = lax.dot_general(zg_ref[...], w2_ref[...], (((1,), (1,)), ((), ())),
                             preferred_element_type=_f32) + b2_ref[...]
        outs[0][...] = hn
        if want_g:
            outs[1][...] = hn * di_ref[...]

    in_specs = [
        pl.BlockSpec((BR, C), lambda i: (i, 0)),
        pl.BlockSpec((C, C), lambda i: (0, 0)),
        pl.BlockSpec((1, C), lambda i: (0, 0)),
    ]
    if want_g:
        in_specs.append(pl.BlockSpec((BR, 1), lambda i: (i, 0)))
    n_out = 2 if want_g else 1
    return pl.pallas_call(
        body,
        grid=(GRID,),
        in_specs=in_specs,
        out_specs=[pl.BlockSpec((BR, C), lambda i: (i, 0))] * n_out,
        out_shape=[jax.ShapeDtypeStruct((NP, C), _f32)] * n_out,
    )


_stage2_g = _make_stage2(True)
_stage2_plain = _make_stage2(False)


def kernel(x, edge_index, batch, wavelet_constructor, W_in, mlp_W1, mlp_b1,
           ln_g, ln_b, mlp_W2, mlp_b2):
    row = jnp.asarray(edge_index[0], jnp.int32)
    col = jnp.asarray(edge_index[1], jnp.int32)
    x_p = jnp.pad(x, ((0, NP - N), (0, 0)))

    perm = jnp.argsort(col, stable=True)
    rs = jnp.pad(row[perm], (0, 2 * W), constant_values=N)
    cs = jnp.pad(col[perm], (0, 2 * W))
    counts = jnp.bincount(col, length=NP)
    ptr = jnp.concatenate([jnp.zeros((1,), jnp.int32),
                           jnp.cumsum(counts).astype(jnp.int32)])
    ts = jnp.pad(ptr[jnp.arange(65) * NB], (0, 15))

    deg_parts = _deg_call(cs[:E])
    dp3 = deg_parts.reshape(2, NP, 1)
    dinv, dinvb, h, g = _input_call(x_p, W_in, dp3)

    outs = []
    for l in range(L):
        w_eff = (wavelet_constructor[l] if l == 0
                 else wavelet_constructor[l] - wavelet_constructor[l - 1])
        ks = _CHECKPOINTS[l]

        def step(carry, _):
            hc, gc = carry
            h2, g2 = _round_call(rs, cs, ts, dinvb, gc, hc)
            return (h2, g2), h2
        (h, g), hs = lax.scan(step, (h, g), None, length=ks[-1])
        cps = {k: hs[k - 1] for k in ks}
        ce = jnp.stack([w_eff[k] for k in ks]).astype(_f32)
        z = _stage1_calls[len(ks)](*[cps[k] for k in ks], ce,
                                   mlp_W1[l], mlp_b1[l].reshape(1, C))
        mu = jnp.mean(z, axis=-1, keepdims=True)
        var = jnp.var(z, axis=-1, keepdims=True)
        z = (z - mu) / jnp.sqrt(var + 1e-5) * ln_g[l] + ln_b[l]
        zg = jax.nn.gelu(z, approximate=False)
        if l < L - 1:
            h, g = _stage2_g(zg, mlp_W2[l], mlp_b2[l].reshape(1, C), dinv)
            outs.append(h[:N])
        else:
            (h_last,) = _stage2_plain(zg, mlp_W2[l], mlp_b2[l].reshape(1, C))
            outs.append(h_last[:N])

    return jnp.concatenate(outs, axis=1)[None]
